# baseline (speedup 1.0000x reference)
import functools

import jax
import jax.numpy as jnp
from jax.experimental import pallas as pl

N_SAMP = 192
N_CH = 3
FLAT = N_SAMP * N_CH
_HI = jax.lax.Precision.HIGHEST


def _body(rgb_ref, nbs_ref, sigma_ref, z_ref,
          comp_ref, w_ref, depth_ref, opac_ref):
    f32 = jnp.float32
    z = z_ref[...]
    sig = jnp.maximum(sigma_ref[...], 0.0)
    r = z.shape[0]

    d = jnp.concatenate(
        [z[:, 1:] - z[:, :-1], jnp.full((r, 1), 1e10, f32)], axis=1)
    ds = d * sig
    alpha = 1.0 - jnp.exp(-ds)

    i0 = jax.lax.broadcasted_iota(jnp.int32, (N_SAMP, N_SAMP), 0)
    i1 = jax.lax.broadcasted_iota(jnp.int32, (N_SAMP, N_SAMP), 1)
    tri = (i0 < i1).astype(f32)
    cums = jax.lax.dot(ds, tri, precision=_HI)
    accum = jnp.exp(-cums)
    w = alpha * accum

    j0 = jax.lax.broadcasted_iota(jnp.int32, (FLAT, N_SAMP), 0)
    k1 = jax.lax.broadcasted_iota(jnp.int32, (FLAT, N_SAMP), 1)
    grp = (j0 // N_CH == k1).astype(f32)
    k0 = jax.lax.broadcasted_iota(jnp.int32, (N_SAMP, FLAT), 0)
    j1 = jax.lax.broadcasted_iota(jnp.int32, (N_SAMP, FLAT), 1)
    rep = (k0 == j1 // N_CH).astype(f32)

    rgbf = rgb_ref[...]
    denom = jnp.ones_like(w)
    srep = jnp.zeros_like(rgbf)
    for i in range(5):
        nb = nbs_ref[i]
        diff = rgbf - nb
        d2 = jax.lax.dot(diff * diff, grp, precision=_HI)
        wi = jnp.exp(-d2)
        denom = denom + wi
        srep = srep + nb * jax.lax.dot(wi, rep, precision=_HI)

    invrep = jax.lax.dot(1.0 / denom, rep, precision=_HI)
    wrep = jax.lax.dot(w, rep, precision=_HI)
    rgb_new = jnp.where(wrep >= 0.01, (rgbf + srep) * invrep, rgbf)

    c0 = jax.lax.broadcasted_iota(jnp.int32, (FLAT, N_CH), 0)
    c1 = jax.lax.broadcasted_iota(jnp.int32, (FLAT, N_CH), 1)
    pick = (c0 % N_CH == c1).astype(f32)
    comp_ref[...] = jax.lax.dot(wrep * rgb_new, pick, precision=_HI)
    w_ref[...] = w
    depth_ref[...] = jnp.sum(w * z, axis=1, keepdims=True)
    opac_ref[...] = jnp.sum(w, axis=1, keepdims=True)


@functools.partial(jax.jit, static_argnames=())
def _run(rgbf, nbs, sigma, z_vals):
    n_ray = rgbf.shape[0]
    blk = 256
    grid = (n_ray // blk,)
    f32 = jnp.float32
    comp, w, depth, opac = pl.pallas_call(
        _body,
        grid=grid,
        in_specs=[
            pl.BlockSpec((blk, FLAT), lambda i: (i, 0)),
            pl.BlockSpec((5, blk, FLAT), lambda i: (0, i, 0)),
            pl.BlockSpec((blk, N_SAMP), lambda i: (i, 0)),
            pl.BlockSpec((blk, N_SAMP), lambda i: (i, 0)),
        ],
        out_specs=[
            pl.BlockSpec((blk, N_CH), lambda i: (i, 0)),
            pl.BlockSpec((blk, N_SAMP), lambda i: (i, 0)),
            pl.BlockSpec((blk, 1), lambda i: (i, 0)),
            pl.BlockSpec((blk, 1), lambda i: (i, 0)),
        ],
        out_shape=[
            jax.ShapeDtypeStruct((n_ray, N_CH), f32),
            jax.ShapeDtypeStruct((n_ray, N_SAMP), f32),
            jax.ShapeDtypeStruct((n_ray, 1), f32),
            jax.ShapeDtypeStruct((n_ray, 1), f32),
        ],
    )(rgbf, nbs, sigma, z_vals)
    return comp, w, depth, opac


def kernel(rgb, neighbor_rgbs, sigma, z_vals, white_bkgd):
    n_ray = rgb.shape[0]
    rgbf = rgb.reshape(n_ray, FLAT)
    nbs = neighbor_rgbs.reshape(5, n_ray, FLAT)
    comp, w, depth, opac = _run(rgbf, nbs, sigma, z_vals)
    opacity = opac[:, 0]
    comp_rgb = jnp.where(white_bkgd, comp + (1.0 - opacity)[:, None], comp)
    return comp_rgb, depth[:, 0], opacity, w
